# baseline (speedup 1.0000x reference)
import functools

import jax
import jax.numpy as jnp
from jax import lax
from jax.experimental import pallas as pl
from jax.experimental.pallas import tpu as pltpu
from jax.experimental.pallas import tpu_sc as plsc

N = 8192
F = 128
KOUT = 50
CHUNK = 16
NCH = N // CHUNK
BR = 128
SENT_BASE = -1.0e30
SENT_STEP = 1.0e26
SENT_CUT = -1.0e29

K64 = 64
GB = 96
NW = 32
RPW = N // NW
NBS = 18
NEG = -3.0e38
POS = 3.0e38


def _ws_body(m_ref, w_ref, b_ref, o_ref):
    o_ref[...] = (
        lax.dot_general(m_ref[...], w_ref[...], (((1,), (1,)), ((), ())),
                        preferred_element_type=jnp.float32)
        + b_ref[...]
    )


def _scores_body(ws_ref, mt_ref, s_ref, cm_ref):
    i = pl.program_id(0)
    acc = lax.dot_general(ws_ref[...], mt_ref[...], (((1,), (0,)), ((), ())),
                          preferred_element_type=jnp.float32)
    rows = i * BR + lax.broadcasted_iota(jnp.int32, (BR, N), 0)
    cols = lax.broadcasted_iota(jnp.int32, (BR, N), 1)
    sent = SENT_BASE - cols.astype(jnp.float32) * SENT_STEP
    masked = jnp.where(cols < rows, acc, sent)
    s_ref[...] = masked
    cm_ref[...] = jnp.max(
        masked.reshape(BR, N // CHUNK, CHUNK), axis=2)


def _masked_scores(mentions, W, b):
    ws = pl.pallas_call(
        _ws_body,
        grid=(N // 1024,),
        in_specs=[
            pl.BlockSpec((1024, F), lambda i: (i, 0)),
            pl.BlockSpec((F, F), lambda i: (0, 0)),
            pl.BlockSpec((1, F), lambda i: (0, 0)),
        ],
        out_specs=pl.BlockSpec((1024, F), lambda i: (i, 0)),
        out_shape=jax.ShapeDtypeStruct((N, F), jnp.float32),
    )(mentions, W, b.reshape(1, F))

    mt = mentions.T
    scores, chunkmax = pl.pallas_call(
        _scores_body,
        grid=(N // BR,),
        in_specs=[
            pl.BlockSpec((BR, F), lambda i: (i, 0)),
            pl.BlockSpec((F, N), lambda i: (0, 0)),
        ],
        out_specs=[
            pl.BlockSpec((BR, N), lambda i: (i, 0)),
            pl.BlockSpec((BR, N // CHUNK), lambda i: (i, 0)),
        ],
        out_shape=[
            jax.ShapeDtypeStruct((N, N), jnp.float32),
            jax.ShapeDtypeStruct((N, N // CHUNK), jnp.float32),
        ],
    )(ws, mt)
    return scores, chunkmax


def _sc_body(scores_hbm, cm_hbm, vals_hbm, idxs_hbm,
             row_v, cm_v, cid_v, vbuf, ibuf, outv, outi, sem_r, sem_c):
    wid = lax.axis_index("s") * 2 + lax.axis_index("c")
    r0 = wid * RPW
    iota = lax.iota(jnp.int32, 16)
    zeros_i = jnp.zeros((16,), jnp.int32)
    negv = jnp.full((16,), NEG, jnp.float32)
    perms = [jnp.bitwise_xor(iota, k) for k in (8, 4, 2, 1)]

    def bf(v, op):
        for pm in perms:
            v = op(v, jnp.take(v, pm))
        return v

    def row_body(rr, carry):
        r = r0 + rr
        rofs = 0
        cofs = 0
        pltpu.sync_copy(scores_hbm.at[r], row_v.at[pl.ds(0, N)])
        pltpu.sync_copy(cm_hbm.at[r], cm_v.at[pl.ds(0, NCH)])

        nc = (r + 15) // 16
        nc_eff = jnp.maximum(nc, 8)
        ncv = (nc_eff + 15) // 16
        use_search = nc_eff >= 128

        def lane_valid(i):
            return (i * 16 + iota) < nc_eff

        def search_fn(_):
            posv = jnp.full((16,), POS, jnp.float32)
            loa, hia = posv, negv
            for i in range(32):
                v = cm_v[pl.ds(cofs + 16 * i, 16)]
                lv = lane_valid(i)
                loa = jnp.minimum(loa, jnp.where(lv, v, POS))
                hia = jnp.maximum(hia, jnp.where(lv, v, NEG))
            lo0 = bf(loa, jnp.minimum)
            hi0 = bf(hia, jnp.maximum)
            hi0 = hi0 + jnp.maximum(jnp.abs(hi0) * 1e-6, 1e-30)

            def bs_body(_i, c):
                lo, hi = c
                midv = 0.5 * (lo + hi)
                acc = zeros_i
                for j in range(32):
                    v = cm_v[pl.ds(cofs + 16 * j, 16)]
                    m = (v >= midv) & lane_valid(j)
                    acc = acc + jnp.where(m, 1, 0)
                okv = bf(acc, jnp.add) >= K64
                return (jnp.where(okv, midv, lo), jnp.where(okv, hi, midv))
            lo, _ = lax.fori_loop(0, NBS, bs_body, (lo0, hi0))
            return lo
        tv = lax.cond(use_search, search_fn, lambda _: negv, 0)

        def f_body(q, p2):
            v = row_v[pl.ds(rofs + 16 * q, 16)]
            m = v >= tv
            plsc.store_compressed(vbuf.at[pl.ds(p2, 16)], v, mask=m)
            plsc.store_compressed(ibuf.at[pl.ds(p2, 16)], iota + 16 * q,
                                  mask=m)
            return p2 + plsc.all_reduce_population_count(m)[0]
        p = lax.fori_loop(0, nc_eff, f_body, jnp.int32(0))

        def cx(a, ai, bb, bi):
            c = a >= bb
            return (jnp.where(c, a, bb), jnp.where(c, ai, bi),
                    jnp.where(c, bb, a), jnp.where(c, bi, ai))

        fifteen = jnp.full((16,), 15, jnp.int32)

        def m_body(q, mc):
            K0, K1, K2, K3, I0, I1, I2, I3 = mc
            base = 16 * q
            lv = (base + iota) < p
            cvm = jnp.where(lv, vbuf[pl.ds(base, 16)], NEG)
            cim = jnp.where(lv, ibuf[pl.ds(base, 16)], 0)
            kminv = jnp.take(K3, fifteen)
            beats = plsc.all_reduce_population_count(cvm > kminv)[0]

            def merge_fn(args):
                K0, K1, K2, K3, I0, I1, I2, I3 = args
                S, SI = plsc.sort_key_val(cvm, cim, descending=True)
                rS = lax.rev(S, (0,))
                rSI = lax.rev(SI, (0,))
                c3 = K3 >= rS
                H3 = jnp.where(c3, K3, rS)
                H3I = jnp.where(c3, I3, rSI)
                A0, A0I, C0, C0I = cx(K0, I0, K2, I2)
                A1, A1I, C1, C1I = cx(K1, I1, H3, H3I)
                B0, B0I, B1, B1I = cx(A0, A0I, A1, A1I)
                B2, B2I, B3, B3I = cx(C0, C0I, C1, C1I)
                K0n, I0n = plsc.sort_key_val(B0, B0I, descending=True)
                K1n, I1n = plsc.sort_key_val(B1, B1I, descending=True)
                K2n, I2n = plsc.sort_key_val(B2, B2I, descending=True)
                K3n, I3n = plsc.sort_key_val(B3, B3I, descending=True)
                return (K0n, K1n, K2n, K3n, I0n, I1n, I2n, I3n)
            return lax.cond(beats > 0, merge_fn, lambda a: a, mc)

        init = (negv, negv, negv, negv, zeros_i, zeros_i, zeros_i, zeros_i)
        res = lax.fori_loop(0, (p + 15) // 16, m_body, init)
        ob = lax.rem(rr, 8) * K64
        for tt in range(4):
            outv[pl.ds(ob + 16 * tt, 16)] = res[tt]
            outi[pl.ds(ob + 16 * tt, 16)] = res[4 + tt]

        @pl.when(lax.rem(rr, 8) == 7)
        def _flush():
            pltpu.sync_copy(outv, vals_hbm.at[pl.ds((r - 7) * K64, 8 * K64)])
            pltpu.sync_copy(outi, idxs_hbm.at[pl.ds((r - 7) * K64, 8 * K64)])
        return carry

    lax.fori_loop(0, RPW, row_body, jnp.int32(0))


def _sc_topk(scores, chunkmax):
    mesh = plsc.VectorSubcoreMesh(core_axis_name="c", subcore_axis_name="s")
    f = pl.kernel(
        _sc_body,
        out_type=[
            jax.ShapeDtypeStruct((N * K64,), jnp.float32),
            jax.ShapeDtypeStruct((N * K64,), jnp.int32),
        ],
        mesh=mesh,
        compiler_params=pltpu.CompilerParams(needs_layout_passes=False),
        scratch_types=[
            pltpu.VMEM((2 * N,), jnp.float32),
            pltpu.VMEM((2 * NCH,), jnp.float32),
            pltpu.VMEM((528,), jnp.int32),
            pltpu.VMEM((8224,), jnp.float32),
            pltpu.VMEM((8224,), jnp.int32),
            pltpu.VMEM((8 * K64,), jnp.float32),
            pltpu.VMEM((8 * K64,), jnp.int32),
            pltpu.SemaphoreType.DMA,
            pltpu.SemaphoreType.DMA,
        ],
    )
    vals_flat, idxs_flat = f(scores, chunkmax)
    return vals_flat.reshape(N, K64), idxs_flat.reshape(N, K64)


def kernel(mentions, first, window_size, W, b):
    scores, chunkmax = _masked_scores(mentions, W, b)
    vals64, idx64 = _sc_topk(scores, chunkmax)
    vals = vals64[:, :KOUT]
    idxs = idx64[:, :KOUT]
    vals = jnp.where(vals < SENT_CUT, -jnp.inf, vals)
    return vals, idxs


# speedup vs baseline: 1.2025x; 1.2025x over previous
import functools

import jax
import jax.numpy as jnp
from jax import lax
from jax.experimental import pallas as pl
from jax.experimental.pallas import tpu as pltpu
from jax.experimental.pallas import tpu_sc as plsc

N = 8192
F = 128
KOUT = 50
CHUNK = 16
NCH = N // CHUNK
BR = 128
SENT_BASE = -1.0e30
SENT_STEP = 1.0e26
SENT_CUT = -1.0e29

K64 = 64
GB = 96
NW = 32
RPW = N // NW
NBS = 18
NEG = -3.0e38
POS = 3.0e38


def _ws_body(m_ref, w_ref, b_ref, o_ref):
    o_ref[...] = (
        lax.dot_general(m_ref[...], w_ref[...], (((1,), (1,)), ((), ())),
                        preferred_element_type=jnp.float32)
        + b_ref[...]
    )


def _scores_body(ws_ref, mt_ref, s_ref, cm_ref):
    i = pl.program_id(0)
    acc = lax.dot_general(ws_ref[...], mt_ref[...], (((1,), (0,)), ((), ())),
                          preferred_element_type=jnp.float32)
    rows = i * BR + lax.broadcasted_iota(jnp.int32, (BR, N), 0)
    cols = lax.broadcasted_iota(jnp.int32, (BR, N), 1)
    sent = SENT_BASE - cols.astype(jnp.float32) * SENT_STEP
    masked = jnp.where(cols < rows, acc, sent)
    s_ref[...] = masked
    cm_ref[...] = jnp.max(
        masked.reshape(BR, N // CHUNK, CHUNK), axis=2)


def _masked_scores(mentions, W, b):
    ws = pl.pallas_call(
        _ws_body,
        grid=(N // 1024,),
        in_specs=[
            pl.BlockSpec((1024, F), lambda i: (i, 0)),
            pl.BlockSpec((F, F), lambda i: (0, 0)),
            pl.BlockSpec((1, F), lambda i: (0, 0)),
        ],
        out_specs=pl.BlockSpec((1024, F), lambda i: (i, 0)),
        out_shape=jax.ShapeDtypeStruct((N, F), jnp.float32),
    )(mentions, W, b.reshape(1, F))

    mt = mentions.T
    scores, chunkmax = pl.pallas_call(
        _scores_body,
        grid=(N // BR,),
        in_specs=[
            pl.BlockSpec((BR, F), lambda i: (i, 0)),
            pl.BlockSpec((F, N), lambda i: (0, 0)),
        ],
        out_specs=[
            pl.BlockSpec((BR, N), lambda i: (i, 0)),
            pl.BlockSpec((BR, N // CHUNK), lambda i: (i, 0)),
        ],
        out_shape=[
            jax.ShapeDtypeStruct((N, N), jnp.float32),
            jax.ShapeDtypeStruct((N, N // CHUNK), jnp.float32),
        ],
    )(ws, mt)
    return scores, chunkmax


def _sc_body(scores_hbm, cm_hbm, vals_hbm, idxs_hbm,
             row_v, cm_v, cid_v, vbuf, ibuf, outv, outi, sem_r, sem_c):
    wid = lax.axis_index("s") * 2 + lax.axis_index("c")
    r0 = wid * RPW
    iota = lax.iota(jnp.int32, 16)
    zeros_i = jnp.zeros((16,), jnp.int32)
    negv = jnp.full((16,), NEG, jnp.float32)
    perms = [jnp.bitwise_xor(iota, k) for k in (8, 4, 2, 1)]

    def bf(v, op):
        for pm in perms:
            v = op(v, jnp.take(v, pm))
        return v

    def row_body(rr, carry):
        r = r0 + rr
        rofs = 0
        cofs = 0
        pltpu.sync_copy(scores_hbm.at[r], row_v.at[pl.ds(0, N)])
        pltpu.sync_copy(cm_hbm.at[r], cm_v.at[pl.ds(0, NCH)])

        nc = (r + 15) // 16
        nc_eff = jnp.maximum(nc, 8)
        ncv = (nc_eff + 15) // 16
        use_search = nc_eff >= 128

        def lane_valid(i):
            return (i * 16 + iota) < nc_eff

        def search_fn(_):
            posv = jnp.full((16,), POS, jnp.float32)

            def mm_body(i, c):
                lo, hi = c
                v = cm_v[pl.ds(cofs + 16 * i, 16)]
                lv = lane_valid(i)
                return (jnp.minimum(lo, jnp.where(lv, v, POS)),
                        jnp.maximum(hi, jnp.where(lv, v, NEG)))
            loa, hia = lax.fori_loop(0, ncv, mm_body, (posv, negv))
            lo0 = bf(loa, jnp.minimum)
            hi0 = bf(hia, jnp.maximum)
            hi0 = hi0 + jnp.maximum(jnp.abs(hi0) * 1e-6, 1e-30)

            def bs_body(_i, c):
                lo, hi = c
                midv = 0.5 * (lo + hi)

                def cnt_body(j, acc):
                    v = cm_v[pl.ds(cofs + 16 * j, 16)]
                    m = (v >= midv) & lane_valid(j)
                    return acc + jnp.where(m, 1, 0)
                acc = lax.fori_loop(0, ncv, cnt_body, zeros_i)
                okv = bf(acc, jnp.add) >= K64
                return (jnp.where(okv, midv, lo), jnp.where(okv, hi, midv))
            lo, _ = lax.fori_loop(0, NBS, bs_body, (lo0, hi0))
            return lo
        tv = lax.cond(use_search, search_fn, lambda _: negv, 0)

        def f_body(q, p2):
            v = row_v[pl.ds(rofs + 16 * q, 16)]
            m = v >= tv
            plsc.store_compressed(vbuf.at[pl.ds(p2, 16)], v, mask=m)
            plsc.store_compressed(ibuf.at[pl.ds(p2, 16)], iota + 16 * q,
                                  mask=m)
            return p2 + plsc.all_reduce_population_count(m)[0]
        p = lax.fori_loop(0, nc_eff, f_body, jnp.int32(0))

        def cx(a, ai, bb, bi):
            c = a >= bb
            return (jnp.where(c, a, bb), jnp.where(c, ai, bi),
                    jnp.where(c, bb, a), jnp.where(c, bi, ai))

        fifteen = jnp.full((16,), 15, jnp.int32)

        def m_body(q, mc):
            K0, K1, K2, K3, I0, I1, I2, I3 = mc
            base = 16 * q
            lv = (base + iota) < p
            cvm = jnp.where(lv, vbuf[pl.ds(base, 16)], NEG)
            cim = jnp.where(lv, ibuf[pl.ds(base, 16)], 0)
            kminv = jnp.take(K3, fifteen)
            beats = plsc.all_reduce_population_count(cvm > kminv)[0]

            def merge_fn(args):
                K0, K1, K2, K3, I0, I1, I2, I3 = args
                S, SI = plsc.sort_key_val(cvm, cim, descending=True)
                rS = lax.rev(S, (0,))
                rSI = lax.rev(SI, (0,))
                c3 = K3 >= rS
                H3 = jnp.where(c3, K3, rS)
                H3I = jnp.where(c3, I3, rSI)
                A0, A0I, C0, C0I = cx(K0, I0, K2, I2)
                A1, A1I, C1, C1I = cx(K1, I1, H3, H3I)
                B0, B0I, B1, B1I = cx(A0, A0I, A1, A1I)
                B2, B2I, B3, B3I = cx(C0, C0I, C1, C1I)
                K0n, I0n = plsc.sort_key_val(B0, B0I, descending=True)
                K1n, I1n = plsc.sort_key_val(B1, B1I, descending=True)
                K2n, I2n = plsc.sort_key_val(B2, B2I, descending=True)
                K3n, I3n = plsc.sort_key_val(B3, B3I, descending=True)
                return (K0n, K1n, K2n, K3n, I0n, I1n, I2n, I3n)
            return lax.cond(beats > 0, merge_fn, lambda a: a, mc)

        init = (negv, negv, negv, negv, zeros_i, zeros_i, zeros_i, zeros_i)
        res = lax.fori_loop(0, (p + 15) // 16, m_body, init)
        ob = lax.rem(rr, 8) * K64
        for tt in range(4):
            outv[pl.ds(ob + 16 * tt, 16)] = res[tt]
            outi[pl.ds(ob + 16 * tt, 16)] = res[4 + tt]

        @pl.when(lax.rem(rr, 8) == 7)
        def _flush():
            pltpu.sync_copy(outv, vals_hbm.at[pl.ds((r - 7) * K64, 8 * K64)])
            pltpu.sync_copy(outi, idxs_hbm.at[pl.ds((r - 7) * K64, 8 * K64)])
        return carry

    lax.fori_loop(0, RPW, row_body, jnp.int32(0))


def _sc_topk(scores, chunkmax):
    mesh = plsc.VectorSubcoreMesh(core_axis_name="c", subcore_axis_name="s")
    f = pl.kernel(
        _sc_body,
        out_type=[
            jax.ShapeDtypeStruct((N * K64,), jnp.float32),
            jax.ShapeDtypeStruct((N * K64,), jnp.int32),
        ],
        mesh=mesh,
        compiler_params=pltpu.CompilerParams(needs_layout_passes=False),
        scratch_types=[
            pltpu.VMEM((2 * N,), jnp.float32),
            pltpu.VMEM((2 * NCH,), jnp.float32),
            pltpu.VMEM((528,), jnp.int32),
            pltpu.VMEM((8224,), jnp.float32),
            pltpu.VMEM((8224,), jnp.int32),
            pltpu.VMEM((8 * K64,), jnp.float32),
            pltpu.VMEM((8 * K64,), jnp.int32),
            pltpu.SemaphoreType.DMA,
            pltpu.SemaphoreType.DMA,
        ],
    )
    vals_flat, idxs_flat = f(scores, chunkmax)
    return vals_flat.reshape(N, K64), idxs_flat.reshape(N, K64)


def kernel(mentions, first, window_size, W, b):
    scores, chunkmax = _masked_scores(mentions, W, b)
    vals64, idx64 = _sc_topk(scores, chunkmax)
    vals = vals64[:, :KOUT]
    idxs = idx64[:, :KOUT]
    vals = jnp.where(vals < SENT_CUT, -jnp.inf, vals)
    return vals, idxs


# speedup vs baseline: 1.3857x; 1.1523x over previous
import functools

import jax
import jax.numpy as jnp
from jax import lax
from jax.experimental import pallas as pl
from jax.experimental.pallas import tpu as pltpu
from jax.experimental.pallas import tpu_sc as plsc

N = 8192
F = 128
KOUT = 50
CHUNK = 16
NCH = N // CHUNK
BR = 128
SENT_BASE = -1.0e30
SENT_STEP = 1.0e26
SENT_CUT = -1.0e29

K64 = 64
GB = 96
NW = 32
RPW = N // NW
NBS = 18
NEG = -3.0e38
POS = 3.0e38


def _ws_body(m_ref, w_ref, b_ref, o_ref):
    o_ref[...] = (
        lax.dot_general(m_ref[...], w_ref[...], (((1,), (1,)), ((), ())),
                        preferred_element_type=jnp.float32)
        + b_ref[...]
    )


def _scores_body(ws_ref, mt_ref, s_ref, cm_ref):
    i = pl.program_id(0)
    acc = lax.dot_general(ws_ref[...], mt_ref[...], (((1,), (0,)), ((), ())),
                          preferred_element_type=jnp.float32)
    rows = i * BR + lax.broadcasted_iota(jnp.int32, (BR, N), 0)
    cols = lax.broadcasted_iota(jnp.int32, (BR, N), 1)
    sent = SENT_BASE - cols.astype(jnp.float32) * SENT_STEP
    masked = jnp.where(cols < rows, acc, sent)
    s_ref[...] = masked
    cm_ref[...] = jnp.max(
        masked.reshape(BR, N // CHUNK, CHUNK), axis=2)


def _masked_scores(mentions, W, b):
    ws = pl.pallas_call(
        _ws_body,
        grid=(N // 1024,),
        in_specs=[
            pl.BlockSpec((1024, F), lambda i: (i, 0)),
            pl.BlockSpec((F, F), lambda i: (0, 0)),
            pl.BlockSpec((1, F), lambda i: (0, 0)),
        ],
        out_specs=pl.BlockSpec((1024, F), lambda i: (i, 0)),
        out_shape=jax.ShapeDtypeStruct((N, F), jnp.float32),
    )(mentions, W, b.reshape(1, F))

    mt = mentions.T
    scores, chunkmax = pl.pallas_call(
        _scores_body,
        grid=(N // BR,),
        in_specs=[
            pl.BlockSpec((BR, F), lambda i: (i, 0)),
            pl.BlockSpec((F, N), lambda i: (0, 0)),
        ],
        out_specs=[
            pl.BlockSpec((BR, N), lambda i: (i, 0)),
            pl.BlockSpec((BR, N // CHUNK), lambda i: (i, 0)),
        ],
        out_shape=[
            jax.ShapeDtypeStruct((N, N), jnp.float32),
            jax.ShapeDtypeStruct((N, N // CHUNK), jnp.float32),
        ],
    )(ws, mt)
    return scores, chunkmax


def _sc_body(scores_hbm, cm_hbm, vals_hbm, idxs_hbm,
             row_v, cm_v, cid_v, vbuf, ibuf, outv, outi, sem_r, sem_c):
    wid = lax.axis_index("s") * 2 + lax.axis_index("c")
    r0 = wid * RPW
    iota = lax.iota(jnp.int32, 16)
    zeros_i = jnp.zeros((16,), jnp.int32)
    negv = jnp.full((16,), NEG, jnp.float32)
    perms = [jnp.bitwise_xor(iota, k) for k in (8, 4, 2, 1)]

    def bf(v, op):
        for pm in perms:
            v = op(v, jnp.take(v, pm))
        return v

    pltpu.async_copy(scores_hbm.at[r0], row_v.at[pl.ds(0, N)], sem_r)
    pltpu.async_copy(cm_hbm.at[r0], cm_v.at[pl.ds(0, NCH)], sem_c)

    def row_body(rr, carry):
        r = r0 + rr
        rofs = lax.rem(rr, 2) * N
        cofs = lax.rem(rr, 2) * NCH
        pltpu.make_async_copy(scores_hbm.at[r],
                              row_v.at[pl.ds(rofs, N)], sem_r).wait()
        pltpu.make_async_copy(cm_hbm.at[r],
                              cm_v.at[pl.ds(cofs, NCH)], sem_c).wait()

        @pl.when(rr + 1 < RPW)
        def _prefetch():
            nofs = lax.rem(rr + 1, 2)
            pltpu.async_copy(scores_hbm.at[r + 1],
                             row_v.at[pl.ds(nofs * N, N)], sem_r)
            pltpu.async_copy(cm_hbm.at[r + 1],
                             cm_v.at[pl.ds(nofs * NCH, NCH)], sem_c)

        nc = (r + 15) // 16
        nc_eff = jnp.maximum(nc, 8)
        ncv = (nc_eff + 15) // 16
        use_search = nc_eff >= 128

        def lane_valid(i):
            return (i * 16 + iota) < nc_eff

        def search_fn(_):
            posv = jnp.full((16,), POS, jnp.float32)

            def mm_body(i, c):
                lo, hi = c
                v = cm_v[pl.ds(cofs + 16 * i, 16)]
                lv = lane_valid(i)
                return (jnp.minimum(lo, jnp.where(lv, v, POS)),
                        jnp.maximum(hi, jnp.where(lv, v, NEG)))
            loa, hia = lax.fori_loop(0, ncv, mm_body, (posv, negv))
            lo0 = bf(loa, jnp.minimum)
            hi0 = bf(hia, jnp.maximum)
            hi0 = hi0 + jnp.maximum(jnp.abs(hi0) * 1e-6, 1e-30)

            def bs_body(_i, c):
                lo, hi = c
                midv = 0.5 * (lo + hi)

                def cnt_body(j, acc):
                    v = cm_v[pl.ds(cofs + 16 * j, 16)]
                    m = (v >= midv) & lane_valid(j)
                    return acc + jnp.where(m, 1, 0)
                acc = lax.fori_loop(0, ncv, cnt_body, zeros_i)
                okv = bf(acc, jnp.add) >= K64
                return (jnp.where(okv, midv, lo), jnp.where(okv, hi, midv))
            lo, _ = lax.fori_loop(0, NBS, bs_body, (lo0, hi0))
            return lo
        tv = lax.cond(use_search, search_fn, lambda _: negv, 0)

        def f_body(q, p2):
            v = row_v[pl.ds(rofs + 16 * q, 16)]
            m = v >= tv
            plsc.store_compressed(vbuf.at[pl.ds(p2, 16)], v, mask=m)
            plsc.store_compressed(ibuf.at[pl.ds(p2, 16)], iota + 16 * q,
                                  mask=m)
            return p2 + plsc.all_reduce_population_count(m)[0]
        p = lax.fori_loop(0, nc_eff, f_body, jnp.int32(0))

        def cx(a, ai, bb, bi):
            c = a >= bb
            return (jnp.where(c, a, bb), jnp.where(c, ai, bi),
                    jnp.where(c, bb, a), jnp.where(c, bi, ai))

        fifteen = jnp.full((16,), 15, jnp.int32)

        def m_body(q, mc):
            K0, K1, K2, K3, I0, I1, I2, I3 = mc
            base = 16 * q
            lv = (base + iota) < p
            cvm = jnp.where(lv, vbuf[pl.ds(base, 16)], NEG)
            cim = jnp.where(lv, ibuf[pl.ds(base, 16)], 0)
            kminv = jnp.take(K3, fifteen)
            beats = plsc.all_reduce_population_count(cvm > kminv)[0]

            def merge_fn(args):
                K0, K1, K2, K3, I0, I1, I2, I3 = args
                S, SI = plsc.sort_key_val(cvm, cim, descending=True)
                rS = lax.rev(S, (0,))
                rSI = lax.rev(SI, (0,))
                c3 = K3 >= rS
                H3 = jnp.where(c3, K3, rS)
                H3I = jnp.where(c3, I3, rSI)
                A0, A0I, C0, C0I = cx(K0, I0, K2, I2)
                A1, A1I, C1, C1I = cx(K1, I1, H3, H3I)
                B0, B0I, B1, B1I = cx(A0, A0I, A1, A1I)
                B2, B2I, B3, B3I = cx(C0, C0I, C1, C1I)
                K0n, I0n = plsc.sort_key_val(B0, B0I, descending=True)
                K1n, I1n = plsc.sort_key_val(B1, B1I, descending=True)
                K2n, I2n = plsc.sort_key_val(B2, B2I, descending=True)
                K3n, I3n = plsc.sort_key_val(B3, B3I, descending=True)
                return (K0n, K1n, K2n, K3n, I0n, I1n, I2n, I3n)
            return lax.cond(beats > 0, merge_fn, lambda a: a, mc)

        init = (negv, negv, negv, negv, zeros_i, zeros_i, zeros_i, zeros_i)
        res = lax.fori_loop(0, (p + 15) // 16, m_body, init)
        ob = lax.rem(rr, 8) * K64
        for tt in range(4):
            outv[pl.ds(ob + 16 * tt, 16)] = res[tt]
            outi[pl.ds(ob + 16 * tt, 16)] = res[4 + tt]

        @pl.when(lax.rem(rr, 8) == 7)
        def _flush():
            pltpu.sync_copy(outv, vals_hbm.at[pl.ds((r - 7) * K64, 8 * K64)])
            pltpu.sync_copy(outi, idxs_hbm.at[pl.ds((r - 7) * K64, 8 * K64)])
        return carry

    lax.fori_loop(0, RPW, row_body, jnp.int32(0))


def _sc_topk(scores, chunkmax):
    mesh = plsc.VectorSubcoreMesh(core_axis_name="c", subcore_axis_name="s")
    f = pl.kernel(
        _sc_body,
        out_type=[
            jax.ShapeDtypeStruct((N * K64,), jnp.float32),
            jax.ShapeDtypeStruct((N * K64,), jnp.int32),
        ],
        mesh=mesh,
        compiler_params=pltpu.CompilerParams(needs_layout_passes=False),
        scratch_types=[
            pltpu.VMEM((2 * N,), jnp.float32),
            pltpu.VMEM((2 * NCH,), jnp.float32),
            pltpu.VMEM((528,), jnp.int32),
            pltpu.VMEM((8224,), jnp.float32),
            pltpu.VMEM((8224,), jnp.int32),
            pltpu.VMEM((8 * K64,), jnp.float32),
            pltpu.VMEM((8 * K64,), jnp.int32),
            pltpu.SemaphoreType.DMA,
            pltpu.SemaphoreType.DMA,
        ],
    )
    vals_flat, idxs_flat = f(scores, chunkmax)
    return vals_flat.reshape(N, K64), idxs_flat.reshape(N, K64)


def kernel(mentions, first, window_size, W, b):
    scores, chunkmax = _masked_scores(mentions, W, b)
    vals64, idx64 = _sc_topk(scores, chunkmax)
    vals = vals64[:, :KOUT]
    idxs = idx64[:, :KOUT]
    vals = jnp.where(vals < SENT_CUT, -jnp.inf, vals)
    return vals, idxs


# speedup vs baseline: 1.3886x; 1.0022x over previous
import functools

import jax
import jax.numpy as jnp
from jax import lax
from jax.experimental import pallas as pl
from jax.experimental.pallas import tpu as pltpu
from jax.experimental.pallas import tpu_sc as plsc

N = 8192
F = 128
KOUT = 50
CHUNK = 16
NCH = N // CHUNK
BR = 128
SENT_BASE = -1.0e30
SENT_STEP = 1.0e26
SENT_CUT = -1.0e29

K64 = 64
NW = 32
RPW = N // NW
NBS = 18
NEG = -3.0e38
POS = 3.0e38


def _ws_body(m_ref, w_ref, b_ref, o_ref):
    o_ref[...] = (
        lax.dot_general(m_ref[...], w_ref[...], (((1,), (1,)), ((), ())),
                        preferred_element_type=jnp.float32)
        + b_ref[...]
    )


def _scores_body(ws_ref, mt_ref, s_ref, cm_ref):
    i = pl.program_id(0)
    acc = lax.dot_general(ws_ref[...], mt_ref[...], (((1,), (0,)), ((), ())),
                          preferred_element_type=jnp.float32)
    rows = i * BR + lax.broadcasted_iota(jnp.int32, (BR, N), 0)
    cols = lax.broadcasted_iota(jnp.int32, (BR, N), 1)
    sent = SENT_BASE - cols.astype(jnp.float32) * SENT_STEP
    masked = jnp.where(cols < rows, acc, sent)
    s_ref[...] = masked
    cm_ref[...] = jnp.max(
        masked.reshape(BR, N // CHUNK, CHUNK), axis=2)


def _masked_scores(mentions, W, b):
    ws = pl.pallas_call(
        _ws_body,
        grid=(N // 1024,),
        in_specs=[
            pl.BlockSpec((1024, F), lambda i: (i, 0)),
            pl.BlockSpec((F, F), lambda i: (0, 0)),
            pl.BlockSpec((1, F), lambda i: (0, 0)),
        ],
        out_specs=pl.BlockSpec((1024, F), lambda i: (i, 0)),
        out_shape=jax.ShapeDtypeStruct((N, F), jnp.float32),
    )(mentions, W, b.reshape(1, F))

    mt = mentions.T
    scores, chunkmax = pl.pallas_call(
        _scores_body,
        grid=(N // BR,),
        in_specs=[
            pl.BlockSpec((BR, F), lambda i: (i, 0)),
            pl.BlockSpec((F, N), lambda i: (0, 0)),
        ],
        out_specs=[
            pl.BlockSpec((BR, N), lambda i: (i, 0)),
            pl.BlockSpec((BR, N // CHUNK), lambda i: (i, 0)),
        ],
        out_shape=[
            jax.ShapeDtypeStruct((N, N), jnp.float32),
            jax.ShapeDtypeStruct((N, N // CHUNK), jnp.float32),
        ],
    )(ws, mt)
    return scores, chunkmax


def _sc_body(scores_hbm, cm_hbm, vals_hbm, idxs_hbm,
             row_v, cm_v, vbuf, ibuf, outv, outi, sem_r, sem_c):
    wid = lax.axis_index("s") * 2 + lax.axis_index("c")
    r0 = wid * RPW
    iota = lax.iota(jnp.int32, 16)
    zeros_i = jnp.zeros((16,), jnp.int32)
    negv = jnp.full((16,), NEG, jnp.float32)
    perms = [jnp.bitwise_xor(iota, k) for k in (8, 4, 2, 1)]

    def bf(v, op):
        for pm in perms:
            v = op(v, jnp.take(v, pm))
        return v

    pltpu.async_copy(scores_hbm.at[r0], row_v.at[pl.ds(0, N)], sem_r)
    pltpu.async_copy(cm_hbm.at[r0], cm_v.at[pl.ds(0, NCH)], sem_c)

    def row_body(rr, carry):
        r = r0 + rr
        rofs = lax.rem(rr, 2) * N
        cofs = lax.rem(rr, 2) * NCH
        pltpu.make_async_copy(scores_hbm.at[r],
                              row_v.at[pl.ds(rofs, N)], sem_r).wait()
        pltpu.make_async_copy(cm_hbm.at[r],
                              cm_v.at[pl.ds(cofs, NCH)], sem_c).wait()

        @pl.when(rr + 1 < RPW)
        def _prefetch():
            nofs = lax.rem(rr + 1, 2)
            pltpu.async_copy(scores_hbm.at[r + 1],
                             row_v.at[pl.ds(nofs * N, N)], sem_r)
            pltpu.async_copy(cm_hbm.at[r + 1],
                             cm_v.at[pl.ds(nofs * NCH, NCH)], sem_c)

        nc = (r + 15) // 16
        nc_eff = jnp.maximum(nc, 8)
        ncv = (nc_eff + 15) // 16
        use_search = nc_eff >= 128

        def lane_valid(i):
            return (i * 16 + iota) < nc_eff

        def search_fn(_):
            posv = jnp.full((16,), POS, jnp.float32)

            def mm_body(i, c):
                lo, hi = c
                v = cm_v[pl.ds(cofs + 16 * i, 16)]
                lv = lane_valid(i)
                return (jnp.minimum(lo, jnp.where(lv, v, POS)),
                        jnp.maximum(hi, jnp.where(lv, v, NEG)))
            loa, hia = lax.fori_loop(0, ncv, mm_body, (posv, negv))
            lo0 = bf(loa, jnp.minimum)
            hi0 = bf(hia, jnp.maximum)
            hi0 = hi0 + jnp.maximum(jnp.abs(hi0) * 1e-6, 1e-30)

            def bs_body(_i, c):
                lo, hi = c
                midv = 0.5 * (lo + hi)

                def cnt_body(j, acc):
                    v = cm_v[pl.ds(cofs + 16 * j, 16)]
                    m = (v >= midv) & lane_valid(j)
                    return acc + jnp.where(m, 1, 0)
                acc = lax.fori_loop(0, ncv, cnt_body, zeros_i)
                okv = bf(acc, jnp.add) >= K64
                return (jnp.where(okv, midv, lo), jnp.where(okv, hi, midv))
            lo, _ = lax.fori_loop(0, NBS, bs_body, (lo0, hi0))
            return lo
        tv = lax.cond(use_search, search_fn, lambda _: negv, 0)

        def f_body(q, p2):
            v = row_v[pl.ds(rofs + 16 * q, 16)]
            m = v >= tv
            plsc.store_compressed(vbuf.at[pl.ds(p2, 16)], v, mask=m)
            plsc.store_compressed(ibuf.at[pl.ds(p2, 16)], iota + 16 * q,
                                  mask=m)
            return p2 + plsc.all_reduce_population_count(m)[0]
        p = lax.fori_loop(0, nc_eff, f_body, jnp.int32(0))

        def cx(a, ai, bb, bi):
            c = a >= bb
            return (jnp.where(c, a, bb), jnp.where(c, ai, bi),
                    jnp.where(c, bb, a), jnp.where(c, bi, ai))

        fifteen = jnp.full((16,), 15, jnp.int32)

        def m_body(q, mc):
            K0, K1, K2, K3, I0, I1, I2, I3 = mc
            base = 16 * q
            lv = (base + iota) < p
            cvm = jnp.where(lv, vbuf[pl.ds(base, 16)], NEG)
            cim = jnp.where(lv, ibuf[pl.ds(base, 16)], 0)
            kminv = jnp.take(K3, fifteen)
            beats = plsc.all_reduce_population_count(cvm > kminv)[0]

            def merge_fn(args):
                K0, K1, K2, K3, I0, I1, I2, I3 = args
                S, SI = plsc.sort_key_val(cvm, cim, descending=True)
                rS = lax.rev(S, (0,))
                rSI = lax.rev(SI, (0,))
                c3 = K3 >= rS
                H3 = jnp.where(c3, K3, rS)
                H3I = jnp.where(c3, I3, rSI)
                A0, A0I, C0, C0I = cx(K0, I0, K2, I2)
                A1, A1I, C1, C1I = cx(K1, I1, H3, H3I)
                B0, B0I, B1, B1I = cx(A0, A0I, A1, A1I)
                B2, B2I, B3, B3I = cx(C0, C0I, C1, C1I)
                K0n, I0n = plsc.sort_key_val(B0, B0I, descending=True)
                K1n, I1n = plsc.sort_key_val(B1, B1I, descending=True)
                K2n, I2n = plsc.sort_key_val(B2, B2I, descending=True)
                K3n, I3n = plsc.sort_key_val(B3, B3I, descending=True)
                return (K0n, K1n, K2n, K3n, I0n, I1n, I2n, I3n)
            return lax.cond(beats > 0, merge_fn, lambda a: a, mc)

        init = (negv, negv, negv, negv, zeros_i, zeros_i, zeros_i, zeros_i)
        res = lax.fori_loop(0, (p + 15) // 16, m_body, init)
        ob = lax.rem(rr, 8) * K64
        for tt in range(4):
            outv[pl.ds(ob + 16 * tt, 16)] = res[tt]
            outi[pl.ds(ob + 16 * tt, 16)] = res[4 + tt]

        @pl.when(lax.rem(rr, 8) == 7)
        def _flush():
            pltpu.sync_copy(outv, vals_hbm.at[pl.ds((r - 7) * K64, 8 * K64)])
            pltpu.sync_copy(outi, idxs_hbm.at[pl.ds((r - 7) * K64, 8 * K64)])
        return carry

    lax.fori_loop(0, RPW, row_body, jnp.int32(0))


def _sc_topk(scores, chunkmax):
    mesh = plsc.VectorSubcoreMesh(core_axis_name="c", subcore_axis_name="s")
    f = pl.kernel(
        _sc_body,
        out_type=[
            jax.ShapeDtypeStruct((N * K64,), jnp.float32),
            jax.ShapeDtypeStruct((N * K64,), jnp.int32),
        ],
        mesh=mesh,
        compiler_params=pltpu.CompilerParams(needs_layout_passes=False),
        scratch_types=[
            pltpu.VMEM((2 * N,), jnp.float32),
            pltpu.VMEM((2 * NCH,), jnp.float32),
            pltpu.VMEM((8224,), jnp.float32),
            pltpu.VMEM((8224,), jnp.int32),
            pltpu.VMEM((8 * K64,), jnp.float32),
            pltpu.VMEM((8 * K64,), jnp.int32),
            pltpu.SemaphoreType.DMA,
            pltpu.SemaphoreType.DMA,
        ],
    )
    vals_flat, idxs_flat = f(scores, chunkmax)
    return vals_flat.reshape(N, K64), idxs_flat.reshape(N, K64)


def kernel(mentions, first, window_size, W, b):
    scores, chunkmax = _masked_scores(mentions, W, b)
    vals64, idx64 = _sc_topk(scores, chunkmax)
    vals = vals64[:, :KOUT]
    idxs = idx64[:, :KOUT]
    vals = jnp.where(vals < SENT_CUT, -jnp.inf, vals)
    return vals, idxs
